# baseline (speedup 1.0000x reference)
import functools

import jax
import jax.numpy as jnp
from jax import lax
from jax.experimental import pallas as pl
from jax.experimental.pallas import tpu as pltpu
from jax.experimental.pallas import tpu_sc as plsc

N = 10000
E = 160000
D = 256
B = 64
L = 256
K = 1024
BETA = 0.25

NB = 41
NP = NB * 256


def _vq_body(agg_ref, x_ref, bcol_ref, w1_ref, w2_ref, cb_ref,
             qn_ref, idxb_ref, starts_ref, counts_ref, loss_ref, cnt_ref,
             first_s, counts_s, loss_s, cnt_s, cn_s):
    i = pl.program_id(0)

    @pl.when(i == 0)
    def _init():
        cb0 = cb_ref[...]
        cn_s[...] = jnp.sum(cb0 * cb0, axis=1)[None, :]
        first_s[...] = jnp.full((1, B), 3.0e7, jnp.float32)
        counts_s[...] = jnp.zeros((1, B), jnp.float32)
        loss_s[...] = jnp.zeros((1, 1), jnp.float32)
        cnt_s[...] = jnp.zeros((1, 1), jnp.float32)

    agg = agg_ref[...]
    xb = x_ref[...]
    h = jnp.maximum(
        jnp.dot(agg, w1_ref[...], preferred_element_type=jnp.float32)
        + jnp.dot(xb, w2_ref[...], preferred_element_type=jnp.float32), 0.0)

    cb = cb_ref[...]
    scores = lax.dot_general(h, cb, (((1,), (1,)), ((), ())),
                             preferred_element_type=jnp.float32)
    zn = jnp.sum(h * h, axis=1, keepdims=True)
    d2 = zn - 2.0 * scores + cn_s[...]
    minv = jnp.min(d2, axis=1, keepdims=True)
    kiota = lax.broadcasted_iota(jnp.int32, (256, K), 1)
    idx = jnp.min(jnp.where(d2 == minv, kiota, K), axis=1, keepdims=True)
    oh = (kiota == idx).astype(jnp.float32)
    q = jnp.dot(oh, cb, preferred_element_type=jnp.float32)
    qn_ref[...] = q
    idxb_ref[...] = jnp.broadcast_to(idx, (256, 128))

    bt = bcol_ref[...]
    b_row = lax.broadcasted_iota(jnp.int32, (1, B), 1)
    oh_b = (bt == b_row)
    grow = i * 256 + lax.broadcasted_iota(jnp.int32, (256, 1), 0)
    growf = grow.astype(jnp.float32)
    blockmin = jnp.min(jnp.where(oh_b, growf, 3.0e7), axis=0, keepdims=True)
    first = jnp.minimum(first_s[...], blockmin)
    first_s[...] = first
    counts_new = counts_s[...] + jnp.sum(
        jnp.where(oh_b, 1.0, 0.0), axis=0, keepdims=True)
    counts_s[...] = counts_new

    start_pn = jnp.sum(jnp.where(oh_b, first, 0.0), axis=1, keepdims=True)
    pos = growf - start_pn
    valid = (pos < float(L)) & (grow < N)
    diff = h - q
    dd = jnp.sum(diff * diff, axis=1, keepdims=True)
    loss_s[...] += jnp.sum(jnp.where(valid, dd, 0.0)).reshape(1, 1)
    cnt_s[...] += jnp.sum(jnp.where(valid, 1.0, 0.0)).reshape(1, 1)

    starts_ref[...] = jnp.minimum(first, float(N)).astype(jnp.int32)
    counts_ref[...] = counts_new.astype(jnp.int32)
    loss_ref[...] = loss_s[...]
    cnt_ref[...] = cnt_s[...]


def _unbatch_body(starts_ref, counts_ref, qn_ref, idxb_ref,
                  out_ref, idxo_ref, qbuf, ibuf, sem1, sem2):
    b = pl.program_id(0)
    s = starts_ref[0, b]
    cnt = counts_ref[0, b]
    sa = (s // 8) * 8
    r = s - sa
    cp1 = pltpu.make_async_copy(qn_ref.at[pl.ds(sa, 264)], qbuf, sem1)
    cp2 = pltpu.make_async_copy(idxb_ref.at[pl.ds(sa, 264)], ibuf, sem2)
    cp1.start()
    cp2.start()
    cp1.wait()
    cp2.wait()
    qrot = pltpu.roll(qbuf[...], 264 - r, 0)[:256]
    irot = pltpu.roll(ibuf[...], 264 - r, 0)[:256]
    prow = lax.broadcasted_iota(jnp.int32, (256, 1), 0)
    m = prow < cnt
    out_ref[...] = jnp.where(m, qrot, 0.0)[None]
    idxo_ref[...] = jnp.where(m, irot[:, 0:1], -1)[None]


W = NP // 32
ACC = W + 8
SCH = 400
FB = 128
SRC_PAD = N
DL_PAD = W


BUF = FB + SCH


def _segsum_body(x_hbm, src_hbm, dst_hbm, zeros_hbm, agg_hbm,
                 acc, src_v, dst_v, msrc, mdst, rows_v, sink, sem):
    c = lax.axis_index("c")
    sid = lax.axis_index("s")
    t = sid * 2 + c
    base = t * W
    iota16 = lax.iota(jnp.int32, 16)
    skew = iota16 * 16

    pltpu.sync_copy(zeros_hbm, acc)

    def flush_block(k):
        pltpu.async_copy(x_hbm.at[msrc.at[pl.ds(k * FB, FB)]],
                         rows_v, sem).wait()
        for g in range(FB // 16):
            rowi = mdst[pl.ds(k * FB + g * 16, 16)]
            j16 = iota16 + g * 16

            def fcol(cc, carry):
                for u in range(8):
                    tt = cc * 8 + u
                    cols = (skew + tt) & (D - 1)
                    vals = plsc.load_gather(rows_v, [j16, cols])
                    plsc.addupdate_scatter(acc, [rowi, cols], vals)
                return carry

            lax.fori_loop(0, D // 8, fcol, 0)

    def sweep(it, cur):
        off = it * SCH
        pltpu.sync_copy(dst_hbm.at[pl.ds(off, SCH)], dst_v)
        pltpu.sync_copy(src_hbm.at[pl.ds(off, SCH)], src_v)

        for g in range(SCH // 16):
            sl = pl.ds(g * 16, 16)
            d = dst_v[sl]
            s16 = src_v[sl]
            dl = d - base
            m = (dl >= 0) & (dl < W)
            inc = plsc.cumsum(jnp.where(m, jnp.int32(1), jnp.int32(0)))
            pos = cur + inc - 1
            plsc.store_scatter(mdst, [pos], dl, mask=m)
            plsc.store_scatter(msrc, [pos], s16, mask=m)
            cur = cur + inc[15]

        nfull = cur // FB
        for k in range(BUF // FB):
            @pl.when(nfull > k)
            def _():
                flush_block(k)

        rem = cur - nfull * FB
        for g in range(FB // 16):
            idxv = iota16 + g * 16
            mrem = idxv < rem
            srcpos = nfull * FB + idxv
            v1 = plsc.load_gather(msrc, [srcpos], mask=mrem)
            v2 = plsc.load_gather(mdst, [srcpos], mask=mrem)
            plsc.store_scatter(msrc, [idxv], v1, mask=mrem)
            plsc.store_scatter(mdst, [idxv], v2, mask=mrem)
        return rem

    cur = lax.fori_loop(0, E // SCH, sweep, jnp.int32(0))
    for g in range(FB // 16):
        idxv = iota16 + g * 16
        mst = idxv >= cur
        plsc.store_scatter(msrc, [idxv], jnp.full((16,), SRC_PAD, jnp.int32),
                           mask=mst)
        plsc.store_scatter(mdst, [idxv], jnp.full((16,), DL_PAD, jnp.int32),
                           mask=mst)
    flush_block(0)
    pltpu.sync_copy(acc.at[pl.ds(0, W)], agg_hbm.at[pl.ds(base, W)])


def _segment_sum(x_pad, src, dst):
    zeros = jnp.zeros((ACC, D), jnp.float32)
    mesh = plsc.VectorSubcoreMesh(core_axis_name="c", subcore_axis_name="s")
    f = pl.kernel(
        _segsum_body,
        out_type=jax.ShapeDtypeStruct((NP, D), jnp.float32),
        mesh=mesh,
        compiler_params=pltpu.CompilerParams(needs_layout_passes=False),
        scratch_types=[
            pltpu.VMEM((ACC, D), jnp.float32),
            pltpu.VMEM((SCH,), jnp.int32),
            pltpu.VMEM((SCH,), jnp.int32),
            pltpu.VMEM((BUF,), jnp.int32),
            pltpu.VMEM((BUF,), jnp.int32),
            pltpu.VMEM((FB, D), jnp.float32),
            pltpu.VMEM((16,), jnp.int32),
            pltpu.SemaphoreType.DMA,
        ],
    )
    return f(x_pad, src, dst, zeros)


def _vq_call(agg, x_pad, bcol, W1, W2, codebook):
    grid = (NB,)
    const = lambda i: (0, 0)
    return pl.pallas_call(
        _vq_body,
        grid=grid,
        in_specs=[
            pl.BlockSpec((256, D), lambda i: (i, 0)),
            pl.BlockSpec((256, D), lambda i: (i, 0)),
            pl.BlockSpec((256, 1), lambda i: (i, 0)),
            pl.BlockSpec((D, D), const),
            pl.BlockSpec((D, D), const),
            pl.BlockSpec((K, D), const),
        ],
        out_specs=[
            pl.BlockSpec((256, D), lambda i: (i, 0)),
            pl.BlockSpec((256, 128), lambda i: (i, 0)),
            pl.BlockSpec((1, B), const),
            pl.BlockSpec((1, B), const),
            pl.BlockSpec((1, 1), const),
            pl.BlockSpec((1, 1), const),
        ],
        out_shape=[
            jax.ShapeDtypeStruct((NP, D), jnp.float32),
            jax.ShapeDtypeStruct((NP, 128), jnp.int32),
            jax.ShapeDtypeStruct((1, B), jnp.int32),
            jax.ShapeDtypeStruct((1, B), jnp.int32),
            jax.ShapeDtypeStruct((1, 1), jnp.float32),
            jax.ShapeDtypeStruct((1, 1), jnp.float32),
        ],
        scratch_shapes=[
            pltpu.VMEM((1, B), jnp.float32),
            pltpu.VMEM((1, B), jnp.float32),
            pltpu.VMEM((1, 1), jnp.float32),
            pltpu.VMEM((1, 1), jnp.float32),
            pltpu.VMEM((1, K), jnp.float32),
        ],
    )(agg, x_pad, bcol, W1, W2, codebook)


def _unbatch_call(starts, counts, qn, idxb):
    return pl.pallas_call(
        _unbatch_body,
        grid=(B,),
        in_specs=[
            pl.BlockSpec(memory_space=pltpu.SMEM),
            pl.BlockSpec(memory_space=pltpu.SMEM),
            pl.BlockSpec(memory_space=pl.ANY),
            pl.BlockSpec(memory_space=pl.ANY),
        ],
        out_specs=[
            pl.BlockSpec((1, L, D), lambda b: (b, 0, 0)),
            pl.BlockSpec((1, L, 1), lambda b: (b, 0, 0)),
        ],
        out_shape=[
            jax.ShapeDtypeStruct((B, L, D), jnp.float32),
            jax.ShapeDtypeStruct((B, L, 1), jnp.int32),
        ],
        scratch_shapes=[
            pltpu.VMEM((264, D), jnp.float32),
            pltpu.VMEM((264, 128), jnp.int32),
            pltpu.SemaphoreType.DMA,
            pltpu.SemaphoreType.DMA,
        ],
    )(starts, counts, qn, idxb)


def kernel(x, edge_index, batch, W1, W2, codebook):
    x_pad = jnp.pad(x, ((0, NP - N), (0, 0)))
    bcol = jnp.pad(batch.astype(jnp.int32), (0, NP - N),
                   constant_values=B).reshape(NP, 1)
    src = edge_index[0].astype(jnp.int32)
    dst = edge_index[1].astype(jnp.int32)

    agg = _segment_sum(x_pad, src, dst)
    qn, idxb, starts, counts, loss, cnt = _vq_call(
        agg, x_pad, bcol, W1, W2, codebook)
    out, idxo = _unbatch_call(starts, counts, qn, idxb)
    commit_loss = (1.0 + BETA) * loss[0, 0] / (cnt[0, 0] * float(D))
    return out, idxo.reshape(B, L), commit_loss
